# baseline (speedup 1.0000x reference)
import functools

import jax
import jax.numpy as jnp
from jax import lax
from jax.experimental import pallas as pl
from jax.experimental.pallas import tpu as pltpu
from jax.experimental.pallas import tpu_sc as plsc

N = 10000
E = 320000
D = 128
NT = 128
L = 3
SURF = 4
CEMB = 32

NC = 2
NS = 16
K = 128
CPT_A = 27
CPT_B = 130
CPTMAX = max(CPT_A, CPT_B)
RPT = 632
RPT_LAST = 528
NPAD = (NS - 1) * RPT + RPT_LAST
NIS = 4
NRS = 3

BLK = 1000


def _sc_body(idx_hbm, m_hbm, zrow_hbm, out_hbm,
             islots, rows, agg, isem, gsem, ssem):
    c = lax.axis_index("c")
    s = lax.axis_index("s")
    pltpu.sync_copy(zrow_hbm, rows.at[pl.ds(0, K)])

    def zblk(j, carry):
        pltpu.sync_copy(rows.at[pl.ds(0, K)],
                        agg.at[pl.ds(s * RPT + j * K, K)])
        return carry

    lax.fori_loop(0, RPT // K, zblk, 0)

    @pl.when(s < NS - 1)
    def _():
        pltpu.sync_copy(rows.at[pl.ds(0, RPT % K)],
                        agg.at[pl.ds(s * RPT + RPT - RPT % K, RPT % K)])

    @pl.when(s == NS - 1)
    def _():
        pltpu.sync_copy(rows.at[pl.ds(0, RPT_LAST % K)],
                        agg.at[pl.ds(s * RPT + RPT_LAST - RPT_LAST % K,
                                     RPT_LAST % K)])

    plsc.subcore_barrier()

    cptc = jnp.where(c == 0, CPT_A, CPT_B)

    def iload(j):
        pltpu.async_copy(idx_hbm.at[c, s, j], islots.at[lax.rem(j, NIS)],
                         isem)

    def iwait():
        pltpu.make_async_copy(idx_hbm.at[c, s, 0], islots.at[0], isem).wait()

    def gstart(i):
        pltpu.async_copy(m_hbm.at[islots.at[lax.rem(i, NIS), 0]],
                         rows.at[pl.ds(lax.rem(i, NRS) * K, K)], gsem)

    def prologue(j, carry):
        iload(j)
        return carry

    lax.fori_loop(0, NIS - 1, prologue, 0)
    iwait()
    iwait()
    gstart(0)
    gstart(1)

    def chunk(i, carry):
        boff = lax.rem(i, NRS) * K

        @pl.when(i >= 1)
        def _():
            pltpu.make_async_copy(m_hbm.at[pl.ds(0, K)],
                                  rows.at[pl.ds(lax.rem(i + 2, NRS) * K, K)],
                                  ssem).wait()

        @pl.when(i + NIS - 1 < cptc)
        def _():
            iload(i + NIS - 1)

        @pl.when(i + 2 < cptc)
        def _():
            iwait()
            gstart(i + 2)

        pltpu.make_async_copy(m_hbm.at[pl.ds(0, K)],
                              rows.at[pl.ds(boff, K)], gsem).wait()
        pltpu.async_copy(rows.at[pl.ds(boff, K)],
                         agg.at[islots.at[lax.rem(i, NIS), 1]], ssem,
                         add=True)

        return carry

    lax.fori_loop(0, cptc, chunk, 0)
    pltpu.make_async_copy(m_hbm.at[pl.ds(0, K)],
                          rows.at[pl.ds(lax.rem(cptc - 1, NRS) * K, K)],
                          ssem).wait()

    plsc.subcore_barrier()

    @pl.when(s < NS - 1)
    def _():
        pltpu.sync_copy(agg.at[pl.ds(s * RPT, RPT)],
                        out_hbm.at[c, pl.ds(s * RPT, RPT)])

    @pl.when(s == NS - 1)
    def _():
        pltpu.sync_copy(agg.at[pl.ds(s * RPT, RPT_LAST)],
                        out_hbm.at[c, pl.ds(s * RPT, RPT_LAST)])


_sc_scatter = functools.partial(
    pl.kernel,
    mesh=plsc.VectorSubcoreMesh(core_axis_name="c", subcore_axis_name="s"),
    out_type=jax.ShapeDtypeStruct((NC, NPAD, D), jnp.float32),
    scratch_types=[
        pltpu.VMEM((NIS, 2, K), jnp.int32),
        pltpu.VMEM((NRS * K, D), jnp.float32),
        pltpu.VMEM_SHARED((NPAD, D), jnp.float32),
        pltpu.SemaphoreType.DMA,
        pltpu.SemaphoreType.DMA,
        pltpu.SemaphoreType.DMA,
    ],
)(_sc_body)


def _pre_body(x_ref, w1t_ref, b1_ref, wg0_ref, h_ref, m_ref):
    h = jnp.dot(x_ref[...], w1t_ref[...], preferred_element_type=jnp.float32)
    h = jnp.maximum(h + b1_ref[...], 0.0)
    h_ref[...] = h
    m_ref[...] = jnp.dot(h, wg0_ref[...], preferred_element_type=jnp.float32)


def _gru_math(parts_ref, h_ref, wiht_ref, whht_ref, bih_ref, bhh_ref):
    agg = parts_ref[0] + parts_ref[1]
    h = h_ref[...]
    gi = jnp.dot(agg, wiht_ref[...], preferred_element_type=jnp.float32)
    gi = gi + bih_ref[...]
    gh = jnp.dot(h, whht_ref[...], preferred_element_type=jnp.float32)
    gh = gh + bhh_ref[...]
    r = jax.nn.sigmoid(gi[:, :D] + gh[:, :D])
    z = jax.nn.sigmoid(gi[:, D:2 * D] + gh[:, D:2 * D])
    n = jnp.tanh(gi[:, 2 * D:] + r * gh[:, 2 * D:])
    return (1.0 - z) * n + z * h


def _gru_body(parts_ref, h_ref, wiht_ref, whht_ref, bih_ref, bhh_ref,
              wgn_ref, hn_ref, mn_ref):
    hn = _gru_math(parts_ref, h_ref, wiht_ref, whht_ref, bih_ref, bhh_ref)
    hn_ref[...] = hn
    mn_ref[...] = jnp.dot(hn, wgn_ref[...], preferred_element_type=jnp.float32)


def _gru_out_body(parts_ref, h_ref, wiht_ref, whht_ref, bih_ref, bhh_ref,
                  w2t_ref, b2_ref, out_ref):
    hn = _gru_math(parts_ref, h_ref, wiht_ref, whht_ref, bih_ref, bhh_ref)
    hn = jnp.maximum(hn, 0.0)
    o = jnp.dot(hn, w2t_ref[...], preferred_element_type=jnp.float32)
    out_ref[...] = o + b2_ref[...]


def _row_spec(shape):
    nd = len(shape)
    if nd == 2:
        return pl.BlockSpec((BLK, shape[1]), lambda i: (i, 0))
    return pl.BlockSpec((shape[0], BLK, shape[2]), lambda i: (0, i, 0))


def _full_spec(shape):
    return pl.BlockSpec(shape, lambda i: tuple(0 for _ in shape))


def _tc_call(body, in_shapes, row_in, out_shapes):
    grid = (N // BLK,)
    in_specs = [_row_spec(s) if r else _full_spec(s)
                for s, r in zip(in_shapes, row_in)]
    out_specs = [_row_spec(s) for s in out_shapes]
    out_shape = [jax.ShapeDtypeStruct(s, jnp.float32) for s in out_shapes]
    if len(out_shapes) == 1:
        out_specs, out_shape = out_specs[0], out_shape[0]
    return pl.pallas_call(body, grid=grid, in_specs=in_specs,
                          out_specs=out_specs, out_shape=out_shape)


def kernel(x, edge_index, W1, b1, Wg, Wih, Whh, bih, bhh, W2, b2):
    w1t = W1.T
    wiht = Wih.T
    whht = Whh.T
    w2t = W2.T
    b1r = b1.reshape(1, D)
    bihr = bih.reshape(1, 3 * D)
    bhhr = bhh.reshape(1, 3 * D)
    b2r = b2.reshape(1, NT)

    src = edge_index[0]
    dst = edge_index[1]
    ecap0 = NS * CPT_A * K
    cap1 = NS * CPT_B * K
    pad1 = ecap0 + cap1 - E
    src0 = src[:ecap0].reshape(NS, CPT_A, K)
    dst0 = dst[:ecap0].reshape(NS, CPT_A, K)
    src1 = jnp.concatenate(
        [src[ecap0:], jnp.zeros((pad1,), jnp.int32)]).reshape(NS, CPT_B, K)
    dst1 = jnp.concatenate(
        [dst[ecap0:], jnp.full((pad1,), N, jnp.int32)]).reshape(NS, CPT_B, K)
    idx0 = jnp.stack([src0, dst0], axis=2)
    idx1 = jnp.stack([src1, dst1], axis=2)
    idx0 = jnp.pad(idx0, ((0, 0), (0, CPTMAX - CPT_A), (0, 0), (0, 0)))
    idx1 = jnp.pad(idx1, ((0, 0), (0, CPTMAX - CPT_B), (0, 0), (0, 0)))
    idx = jnp.stack([idx0, idx1], axis=0)
    zrow = jnp.zeros((K, D), jnp.float32)

    pre = _tc_call(_pre_body,
                   [(N, D), (D, D), (1, D), (D, D)],
                   [True, False, False, False],
                   [(N, D), (N, D)])
    h, m = pre(x, w1t, b1r, Wg[0])

    gru = _tc_call(_gru_body,
                   [(NC, NPAD, D), (N, D), (D, 3 * D), (D, 3 * D),
                    (1, 3 * D), (1, 3 * D), (D, D)],
                   [True, True, False, False, False, False, False],
                   [(N, D), (N, D)])
    gru_out = _tc_call(_gru_out_body,
                       [(NC, NPAD, D), (N, D), (D, 3 * D), (D, 3 * D),
                        (1, 3 * D), (1, 3 * D), (D, NT), (1, NT)],
                       [True, True, False, False, False, False, False, False],
                       [(N, NT)])

    for i in range(L):
        parts = _sc_scatter(idx, m, zrow)
        if i < L - 1:
            h, m = gru(parts, h, wiht, whht, bihr, bhhr, Wg[i + 1])
        else:
            out = gru_out(parts, h, wiht, whht, bihr, bhhr, w2t, b2r)
    return out.reshape(N, SURF, CEMB)


# speedup vs baseline: 1.0927x; 1.0927x over previous
import functools

import jax
import jax.numpy as jnp
from jax import lax
from jax.experimental import pallas as pl
from jax.experimental.pallas import tpu as pltpu
from jax.experimental.pallas import tpu_sc as plsc

N = 10000
E = 320000
D = 128
NT = 128
L = 3
SURF = 4
CEMB = 32

NC = 2
NS = 16
K = 128
CPT_A = 28
CPT_B = 129
CPTMAX = max(CPT_A, CPT_B)
RPT = 632
RPT_LAST = 528
NPAD = (NS - 1) * RPT + RPT_LAST
NIS = 4
NRS = 3

BLK = 1000


def _sc_body(idx_hbm, m_hbm, zrow_hbm, out_hbm,
             islots, rows, agg, isem, gsem, ssem):
    c = lax.axis_index("c")
    s = lax.axis_index("s")
    pltpu.sync_copy(zrow_hbm, rows.at[pl.ds(0, K)])

    def zblk(j, carry):
        pltpu.sync_copy(rows.at[pl.ds(0, K)],
                        agg.at[pl.ds(s * RPT + j * K, K)])
        return carry

    lax.fori_loop(0, RPT // K, zblk, 0)

    @pl.when(s < NS - 1)
    def _():
        pltpu.sync_copy(rows.at[pl.ds(0, RPT % K)],
                        agg.at[pl.ds(s * RPT + RPT - RPT % K, RPT % K)])

    @pl.when(s == NS - 1)
    def _():
        pltpu.sync_copy(rows.at[pl.ds(0, RPT_LAST % K)],
                        agg.at[pl.ds(s * RPT + RPT_LAST - RPT_LAST % K,
                                     RPT_LAST % K)])

    plsc.subcore_barrier()

    cptc = jnp.where(c == 0, CPT_A, CPT_B)

    def iload(j):
        pltpu.async_copy(idx_hbm.at[c, s, j], islots.at[lax.rem(j, NIS)],
                         isem)

    def iwait():
        pltpu.make_async_copy(idx_hbm.at[c, s, 0], islots.at[0], isem).wait()

    def gstart(i):
        pltpu.async_copy(m_hbm.at[islots.at[lax.rem(i, NIS), 0]],
                         rows.at[pl.ds(lax.rem(i, NRS) * K, K)], gsem)

    def prologue(j, carry):
        iload(j)
        return carry

    lax.fori_loop(0, NIS - 1, prologue, 0)
    iwait()
    iwait()
    gstart(0)
    gstart(1)

    def chunk(i, carry):
        boff = lax.rem(i, NRS) * K

        @pl.when(i >= 1)
        def _():
            pltpu.make_async_copy(m_hbm.at[pl.ds(0, K)],
                                  rows.at[pl.ds(lax.rem(i + 2, NRS) * K, K)],
                                  ssem).wait()

        @pl.when(i + NIS - 1 < cptc)
        def _():
            iload(i + NIS - 1)

        @pl.when(i + 2 < cptc)
        def _():
            iwait()
            gstart(i + 2)

        pltpu.make_async_copy(m_hbm.at[pl.ds(0, K)],
                              rows.at[pl.ds(boff, K)], gsem).wait()
        pltpu.async_copy(rows.at[pl.ds(boff, K)],
                         agg.at[islots.at[lax.rem(i, NIS), 1]], ssem,
                         add=True)

        return carry

    lax.fori_loop(0, cptc, chunk, 0)
    pltpu.make_async_copy(m_hbm.at[pl.ds(0, K)],
                          rows.at[pl.ds(lax.rem(cptc - 1, NRS) * K, K)],
                          ssem).wait()

    plsc.subcore_barrier()

    @pl.when(s < NS - 1)
    def _():
        pltpu.sync_copy(agg.at[pl.ds(s * RPT, RPT)],
                        out_hbm.at[c, pl.ds(s * RPT, RPT)])

    @pl.when(s == NS - 1)
    def _():
        pltpu.sync_copy(agg.at[pl.ds(s * RPT, RPT_LAST)],
                        out_hbm.at[c, pl.ds(s * RPT, RPT_LAST)])


_sc_scatter = functools.partial(
    pl.kernel,
    mesh=plsc.VectorSubcoreMesh(core_axis_name="c", subcore_axis_name="s"),
    out_type=jax.ShapeDtypeStruct((NC, NPAD, D), jnp.float32),
    scratch_types=[
        pltpu.VMEM((NIS, 2, K), jnp.int32),
        pltpu.VMEM((NRS * K, D), jnp.float32),
        pltpu.VMEM_SHARED((NPAD, D), jnp.float32),
        pltpu.SemaphoreType.DMA,
        pltpu.SemaphoreType.DMA,
        pltpu.SemaphoreType.DMA,
    ],
)(_sc_body)


def _pre_body(x_ref, w1t_ref, b1_ref, wg0_ref, h_ref, m_ref):
    h = jnp.dot(x_ref[...], w1t_ref[...], preferred_element_type=jnp.float32)
    h = jnp.maximum(h + b1_ref[...], 0.0)
    h_ref[...] = h
    m_ref[...] = jnp.dot(h, wg0_ref[...], preferred_element_type=jnp.float32)


def _gru_math(parts_ref, h_ref, wiht_ref, whht_ref, bih_ref, bhh_ref):
    agg = parts_ref[0] + parts_ref[1]
    h = h_ref[...]
    gi = jnp.dot(agg, wiht_ref[...], preferred_element_type=jnp.float32)
    gi = gi + bih_ref[...]
    gh = jnp.dot(h, whht_ref[...], preferred_element_type=jnp.float32)
    gh = gh + bhh_ref[...]
    r = jax.nn.sigmoid(gi[:, :D] + gh[:, :D])
    z = jax.nn.sigmoid(gi[:, D:2 * D] + gh[:, D:2 * D])
    n = jnp.tanh(gi[:, 2 * D:] + r * gh[:, 2 * D:])
    return (1.0 - z) * n + z * h


def _gru_body(parts_ref, h_ref, wiht_ref, whht_ref, bih_ref, bhh_ref,
              wgn_ref, hn_ref, mn_ref):
    hn = _gru_math(parts_ref, h_ref, wiht_ref, whht_ref, bih_ref, bhh_ref)
    hn_ref[...] = hn
    mn_ref[...] = jnp.dot(hn, wgn_ref[...], preferred_element_type=jnp.float32)


def _gru_out_body(parts_ref, h_ref, wiht_ref, whht_ref, bih_ref, bhh_ref,
                  w2t_ref, b2_ref, out_ref):
    hn = _gru_math(parts_ref, h_ref, wiht_ref, whht_ref, bih_ref, bhh_ref)
    hn = jnp.maximum(hn, 0.0)
    o = jnp.dot(hn, w2t_ref[...], preferred_element_type=jnp.float32)
    out_ref[...] = o + b2_ref[...]


def _row_spec(shape):
    nd = len(shape)
    if nd == 2:
        return pl.BlockSpec((BLK, shape[1]), lambda i: (i, 0))
    return pl.BlockSpec((shape[0], BLK, shape[2]), lambda i: (0, i, 0))


def _full_spec(shape):
    return pl.BlockSpec(shape, lambda i: tuple(0 for _ in shape))


def _tc_call(body, in_shapes, row_in, out_shapes):
    grid = (N // BLK,)
    in_specs = [_row_spec(s) if r else _full_spec(s)
                for s, r in zip(in_shapes, row_in)]
    out_specs = [_row_spec(s) for s in out_shapes]
    out_shape = [jax.ShapeDtypeStruct(s, jnp.float32) for s in out_shapes]
    if len(out_shapes) == 1:
        out_specs, out_shape = out_specs[0], out_shape[0]
    return pl.pallas_call(body, grid=grid, in_specs=in_specs,
                          out_specs=out_specs, out_shape=out_shape)


def kernel(x, edge_index, W1, b1, Wg, Wih, Whh, bih, bhh, W2, b2):
    w1t = W1.T
    wiht = Wih.T
    whht = Whh.T
    w2t = W2.T
    b1r = b1.reshape(1, D)
    bihr = bih.reshape(1, 3 * D)
    bhhr = bhh.reshape(1, 3 * D)
    b2r = b2.reshape(1, NT)

    src = edge_index[0]
    dst = edge_index[1]
    ecap0 = NS * CPT_A * K
    cap1 = NS * CPT_B * K
    pad1 = ecap0 + cap1 - E
    src0 = src[:ecap0].reshape(NS, CPT_A, K)
    dst0 = dst[:ecap0].reshape(NS, CPT_A, K)
    src1 = jnp.concatenate(
        [src[ecap0:], jnp.zeros((pad1,), jnp.int32)]).reshape(NS, CPT_B, K)
    dst1 = jnp.concatenate(
        [dst[ecap0:], jnp.full((pad1,), N, jnp.int32)]).reshape(NS, CPT_B, K)
    idx0 = jnp.stack([src0, dst0], axis=2)
    idx1 = jnp.stack([src1, dst1], axis=2)
    idx0 = jnp.pad(idx0, ((0, 0), (0, CPTMAX - CPT_A), (0, 0), (0, 0)))
    idx1 = jnp.pad(idx1, ((0, 0), (0, CPTMAX - CPT_B), (0, 0), (0, 0)))
    idx = jnp.stack([idx0, idx1], axis=0)
    zrow = jnp.zeros((K, D), jnp.float32)

    pre = _tc_call(_pre_body,
                   [(N, D), (D, D), (1, D), (D, D)],
                   [True, False, False, False],
                   [(N, D), (N, D)])
    h, m = pre(x, w1t, b1r, Wg[0])

    gru = _tc_call(_gru_body,
                   [(NC, NPAD, D), (N, D), (D, 3 * D), (D, 3 * D),
                    (1, 3 * D), (1, 3 * D), (D, D)],
                   [True, True, False, False, False, False, False],
                   [(N, D), (N, D)])
    gru_out = _tc_call(_gru_out_body,
                       [(NC, NPAD, D), (N, D), (D, 3 * D), (D, 3 * D),
                        (1, 3 * D), (1, 3 * D), (D, NT), (1, NT)],
                       [True, True, False, False, False, False, False, False],
                       [(N, NT)])

    for i in range(L):
        parts = _sc_scatter(idx, m, zrow)
        if i < L - 1:
            h, m = gru(parts, h, wiht, whht, bihr, bhhr, Wg[i + 1])
        else:
            out = gru_out(parts, h, wiht, whht, bihr, bhhr, w2t, b2r)
    return out.reshape(N, SURF, CEMB)


# speedup vs baseline: 1.1100x; 1.0159x over previous
import functools

import jax
import jax.numpy as jnp
from jax import lax
from jax.experimental import pallas as pl
from jax.experimental.pallas import tpu as pltpu
from jax.experimental.pallas import tpu_sc as plsc

N = 10000
E = 320000
D = 128
NT = 128
L = 3
SURF = 4
CEMB = 32

NC = 2
NS = 16
K = 128
CPT_A = 28
CPT_B = 129
CPTMAX = max(CPT_A, CPT_B)
RPT = 632
RPT_LAST = 528
NPAD = (NS - 1) * RPT + RPT_LAST
NIS = 4
NRS = 3

BLK = 2000


def _sc_body(idx_hbm, m_hbm, zrow_hbm, out_hbm,
             islots, rows, agg, isem, gsem, ssem):
    c = lax.axis_index("c")
    s = lax.axis_index("s")
    pltpu.sync_copy(zrow_hbm, rows.at[pl.ds(0, K)])

    def zblk(j, carry):
        pltpu.sync_copy(rows.at[pl.ds(0, K)],
                        agg.at[pl.ds(s * RPT + j * K, K)])
        return carry

    lax.fori_loop(0, RPT // K, zblk, 0)

    @pl.when(s < NS - 1)
    def _():
        pltpu.sync_copy(rows.at[pl.ds(0, RPT % K)],
                        agg.at[pl.ds(s * RPT + RPT - RPT % K, RPT % K)])

    @pl.when(s == NS - 1)
    def _():
        pltpu.sync_copy(rows.at[pl.ds(0, RPT_LAST % K)],
                        agg.at[pl.ds(s * RPT + RPT_LAST - RPT_LAST % K,
                                     RPT_LAST % K)])

    plsc.subcore_barrier()

    cptc = jnp.where(c == 0, CPT_A, CPT_B)

    def iload(j):
        pltpu.async_copy(idx_hbm.at[c, s, j], islots.at[lax.rem(j, NIS)],
                         isem)

    def iwait():
        pltpu.make_async_copy(idx_hbm.at[c, s, 0], islots.at[0], isem).wait()

    def gstart(i):
        pltpu.async_copy(m_hbm.at[islots.at[lax.rem(i, NIS), 0]],
                         rows.at[pl.ds(lax.rem(i, NRS) * K, K)], gsem)

    def prologue(j, carry):
        iload(j)
        return carry

    lax.fori_loop(0, NIS - 1, prologue, 0)
    iwait()
    iwait()
    gstart(0)
    gstart(1)

    def chunk(i, carry):
        boff = lax.rem(i, NRS) * K

        @pl.when(i >= 1)
        def _():
            pltpu.make_async_copy(m_hbm.at[pl.ds(0, K)],
                                  rows.at[pl.ds(lax.rem(i + 2, NRS) * K, K)],
                                  ssem).wait()

        @pl.when(i + NIS - 1 < cptc)
        def _():
            iload(i + NIS - 1)

        @pl.when(i + 2 < cptc)
        def _():
            iwait()
            gstart(i + 2)

        pltpu.make_async_copy(m_hbm.at[pl.ds(0, K)],
                              rows.at[pl.ds(boff, K)], gsem).wait()
        pltpu.async_copy(rows.at[pl.ds(boff, K)],
                         agg.at[islots.at[lax.rem(i, NIS), 1]], ssem,
                         add=True)

        return carry

    lax.fori_loop(0, cptc, chunk, 0)
    pltpu.make_async_copy(m_hbm.at[pl.ds(0, K)],
                          rows.at[pl.ds(lax.rem(cptc - 1, NRS) * K, K)],
                          ssem).wait()

    plsc.subcore_barrier()

    @pl.when(s < NS - 1)
    def _():
        pltpu.sync_copy(agg.at[pl.ds(s * RPT, RPT)],
                        out_hbm.at[c, pl.ds(s * RPT, RPT)])

    @pl.when(s == NS - 1)
    def _():
        pltpu.sync_copy(agg.at[pl.ds(s * RPT, RPT_LAST)],
                        out_hbm.at[c, pl.ds(s * RPT, RPT_LAST)])


_sc_scatter = functools.partial(
    pl.kernel,
    mesh=plsc.VectorSubcoreMesh(core_axis_name="c", subcore_axis_name="s"),
    out_type=jax.ShapeDtypeStruct((NC, NPAD, D), jnp.float32),
    scratch_types=[
        pltpu.VMEM((NIS, 2, K), jnp.int32),
        pltpu.VMEM((NRS * K, D), jnp.float32),
        pltpu.VMEM_SHARED((NPAD, D), jnp.float32),
        pltpu.SemaphoreType.DMA,
        pltpu.SemaphoreType.DMA,
        pltpu.SemaphoreType.DMA,
    ],
)(_sc_body)


def _pre_body(x_ref, w1t_ref, b1_ref, wg0_ref, h_ref, m_ref):
    h = jnp.dot(x_ref[...], w1t_ref[...], preferred_element_type=jnp.float32)
    h = jnp.maximum(h + b1_ref[...], 0.0)
    h_ref[...] = h
    m_ref[...] = jnp.dot(h, wg0_ref[...], preferred_element_type=jnp.float32)


def _gru_math(parts_ref, h_ref, wiht_ref, whht_ref, bih_ref, bhh_ref):
    agg = parts_ref[0] + parts_ref[1]
    h = h_ref[...]
    gi = jnp.dot(agg, wiht_ref[...], preferred_element_type=jnp.float32)
    gi = gi + bih_ref[...]
    gh = jnp.dot(h, whht_ref[...], preferred_element_type=jnp.float32)
    gh = gh + bhh_ref[...]
    r = jax.nn.sigmoid(gi[:, :D] + gh[:, :D])
    z = jax.nn.sigmoid(gi[:, D:2 * D] + gh[:, D:2 * D])
    n = jnp.tanh(gi[:, 2 * D:] + r * gh[:, 2 * D:])
    return (1.0 - z) * n + z * h


def _gru_body(parts_ref, h_ref, wiht_ref, whht_ref, bih_ref, bhh_ref,
              wgn_ref, hn_ref, mn_ref):
    hn = _gru_math(parts_ref, h_ref, wiht_ref, whht_ref, bih_ref, bhh_ref)
    hn_ref[...] = hn
    mn_ref[...] = jnp.dot(hn, wgn_ref[...], preferred_element_type=jnp.float32)


def _gru_out_body(parts_ref, h_ref, wiht_ref, whht_ref, bih_ref, bhh_ref,
                  w2t_ref, b2_ref, out_ref):
    hn = _gru_math(parts_ref, h_ref, wiht_ref, whht_ref, bih_ref, bhh_ref)
    hn = jnp.maximum(hn, 0.0)
    o = jnp.dot(hn, w2t_ref[...], preferred_element_type=jnp.float32)
    out_ref[...] = o + b2_ref[...]


def _row_spec(shape):
    nd = len(shape)
    if nd == 2:
        return pl.BlockSpec((BLK, shape[1]), lambda i: (i, 0))
    return pl.BlockSpec((shape[0], BLK, shape[2]), lambda i: (0, i, 0))


def _full_spec(shape):
    return pl.BlockSpec(shape, lambda i: tuple(0 for _ in shape))


def _tc_call(body, in_shapes, row_in, out_shapes):
    grid = (N // BLK,)
    in_specs = [_row_spec(s) if r else _full_spec(s)
                for s, r in zip(in_shapes, row_in)]
    out_specs = [_row_spec(s) for s in out_shapes]
    out_shape = [jax.ShapeDtypeStruct(s, jnp.float32) for s in out_shapes]
    if len(out_shapes) == 1:
        out_specs, out_shape = out_specs[0], out_shape[0]
    return pl.pallas_call(body, grid=grid, in_specs=in_specs,
                          out_specs=out_specs, out_shape=out_shape)


def kernel(x, edge_index, W1, b1, Wg, Wih, Whh, bih, bhh, W2, b2):
    w1t = W1.T
    wiht = Wih.T
    whht = Whh.T
    w2t = W2.T
    b1r = b1.reshape(1, D)
    bihr = bih.reshape(1, 3 * D)
    bhhr = bhh.reshape(1, 3 * D)
    b2r = b2.reshape(1, NT)

    src = edge_index[0]
    dst = edge_index[1]
    ecap0 = NS * CPT_A * K
    cap1 = NS * CPT_B * K
    pad1 = ecap0 + cap1 - E
    src0 = src[:ecap0].reshape(NS, CPT_A, K)
    dst0 = dst[:ecap0].reshape(NS, CPT_A, K)
    src1 = jnp.concatenate(
        [src[ecap0:], jnp.zeros((pad1,), jnp.int32)]).reshape(NS, CPT_B, K)
    dst1 = jnp.concatenate(
        [dst[ecap0:], jnp.full((pad1,), N, jnp.int32)]).reshape(NS, CPT_B, K)
    idx0 = jnp.stack([src0, dst0], axis=2)
    idx1 = jnp.stack([src1, dst1], axis=2)
    idx0 = jnp.pad(idx0, ((0, 0), (0, CPTMAX - CPT_A), (0, 0), (0, 0)))
    idx1 = jnp.pad(idx1, ((0, 0), (0, CPTMAX - CPT_B), (0, 0), (0, 0)))
    idx = jnp.stack([idx0, idx1], axis=0)
    zrow = jnp.zeros((K, D), jnp.float32)

    pre = _tc_call(_pre_body,
                   [(N, D), (D, D), (1, D), (D, D)],
                   [True, False, False, False],
                   [(N, D), (N, D)])
    h, m = pre(x, w1t, b1r, Wg[0])

    gru = _tc_call(_gru_body,
                   [(NC, NPAD, D), (N, D), (D, 3 * D), (D, 3 * D),
                    (1, 3 * D), (1, 3 * D), (D, D)],
                   [True, True, False, False, False, False, False],
                   [(N, D), (N, D)])
    gru_out = _tc_call(_gru_out_body,
                       [(NC, NPAD, D), (N, D), (D, 3 * D), (D, 3 * D),
                        (1, 3 * D), (1, 3 * D), (D, NT), (1, NT)],
                       [True, True, False, False, False, False, False, False],
                       [(N, NT)])

    for i in range(L):
        parts = _sc_scatter(idx, m, zrow)
        if i < L - 1:
            h, m = gru(parts, h, wiht, whht, bihr, bhhr, Wg[i + 1])
        else:
            out = gru_out(parts, h, wiht, whht, bihr, bhhr, w2t, b2r)
    return out.reshape(N, SURF, CEMB)


# speedup vs baseline: 1.1141x; 1.0037x over previous
import functools

import jax
import jax.numpy as jnp
from jax import lax
from jax.experimental import pallas as pl
from jax.experimental.pallas import tpu as pltpu
from jax.experimental.pallas import tpu_sc as plsc

N = 10000
E = 320000
D = 128
NT = 128
L = 3
SURF = 4
CEMB = 32

NC = 2
NS = 16
K = 128
CPT_A = 28
CPT_B = 129
CPTMAX = max(CPT_A, CPT_B)
RPT = 632
RPT_LAST = 528
NPAD = (NS - 1) * RPT + RPT_LAST
NIS = 4
NRS = 3

BLK = 5000


def _sc_body(idx_hbm, m_hbm, zrow_hbm, out_hbm,
             islots, rows, agg, isem, gsem, ssem):
    c = lax.axis_index("c")
    s = lax.axis_index("s")
    pltpu.sync_copy(zrow_hbm, rows.at[pl.ds(0, K)])

    def zblk(j, carry):
        pltpu.sync_copy(rows.at[pl.ds(0, K)],
                        agg.at[pl.ds(s * RPT + j * K, K)])
        return carry

    lax.fori_loop(0, RPT // K, zblk, 0)

    @pl.when(s < NS - 1)
    def _():
        pltpu.sync_copy(rows.at[pl.ds(0, RPT % K)],
                        agg.at[pl.ds(s * RPT + RPT - RPT % K, RPT % K)])

    @pl.when(s == NS - 1)
    def _():
        pltpu.sync_copy(rows.at[pl.ds(0, RPT_LAST % K)],
                        agg.at[pl.ds(s * RPT + RPT_LAST - RPT_LAST % K,
                                     RPT_LAST % K)])

    plsc.subcore_barrier()

    cptc = jnp.where(c == 0, CPT_A, CPT_B)

    def iload(j):
        pltpu.async_copy(idx_hbm.at[c, s, j], islots.at[lax.rem(j, NIS)],
                         isem)

    def iwait():
        pltpu.make_async_copy(idx_hbm.at[c, s, 0], islots.at[0], isem).wait()

    def gstart(i):
        pltpu.async_copy(m_hbm.at[islots.at[lax.rem(i, NIS), 0]],
                         rows.at[pl.ds(lax.rem(i, NRS) * K, K)], gsem)

    def prologue(j, carry):
        iload(j)
        return carry

    lax.fori_loop(0, NIS - 1, prologue, 0)
    iwait()
    iwait()
    gstart(0)
    gstart(1)

    def chunk(i, carry):
        boff = lax.rem(i, NRS) * K

        @pl.when(i >= 1)
        def _():
            pltpu.make_async_copy(m_hbm.at[pl.ds(0, K)],
                                  rows.at[pl.ds(lax.rem(i + 2, NRS) * K, K)],
                                  ssem).wait()

        @pl.when(i + NIS - 1 < cptc)
        def _():
            iload(i + NIS - 1)

        @pl.when(i + 2 < cptc)
        def _():
            iwait()
            gstart(i + 2)

        pltpu.make_async_copy(m_hbm.at[pl.ds(0, K)],
                              rows.at[pl.ds(boff, K)], gsem).wait()
        pltpu.async_copy(rows.at[pl.ds(boff, K)],
                         agg.at[islots.at[lax.rem(i, NIS), 1]], ssem,
                         add=True)

        return carry

    lax.fori_loop(0, cptc, chunk, 0)
    pltpu.make_async_copy(m_hbm.at[pl.ds(0, K)],
                          rows.at[pl.ds(lax.rem(cptc - 1, NRS) * K, K)],
                          ssem).wait()

    plsc.subcore_barrier()

    @pl.when(s < NS - 1)
    def _():
        pltpu.sync_copy(agg.at[pl.ds(s * RPT, RPT)],
                        out_hbm.at[c, pl.ds(s * RPT, RPT)])

    @pl.when(s == NS - 1)
    def _():
        pltpu.sync_copy(agg.at[pl.ds(s * RPT, RPT_LAST)],
                        out_hbm.at[c, pl.ds(s * RPT, RPT_LAST)])


_sc_scatter = functools.partial(
    pl.kernel,
    mesh=plsc.VectorSubcoreMesh(core_axis_name="c", subcore_axis_name="s"),
    out_type=jax.ShapeDtypeStruct((NC, NPAD, D), jnp.float32),
    scratch_types=[
        pltpu.VMEM((NIS, 2, K), jnp.int32),
        pltpu.VMEM((NRS * K, D), jnp.float32),
        pltpu.VMEM_SHARED((NPAD, D), jnp.float32),
        pltpu.SemaphoreType.DMA,
        pltpu.SemaphoreType.DMA,
        pltpu.SemaphoreType.DMA,
    ],
)(_sc_body)


def _pre_body(x_ref, w1t_ref, b1_ref, wg0_ref, h_ref, m_ref):
    h = jnp.dot(x_ref[...], w1t_ref[...], preferred_element_type=jnp.float32)
    h = jnp.maximum(h + b1_ref[...], 0.0)
    h_ref[...] = h
    m_ref[...] = jnp.dot(h, wg0_ref[...], preferred_element_type=jnp.float32)


def _gru_math(parts_ref, h_ref, wiht_ref, whht_ref, bih_ref, bhh_ref):
    agg = parts_ref[0] + parts_ref[1]
    h = h_ref[...]
    gi = jnp.dot(agg, wiht_ref[...], preferred_element_type=jnp.float32)
    gi = gi + bih_ref[...]
    gh = jnp.dot(h, whht_ref[...], preferred_element_type=jnp.float32)
    gh = gh + bhh_ref[...]
    r = jax.nn.sigmoid(gi[:, :D] + gh[:, :D])
    z = jax.nn.sigmoid(gi[:, D:2 * D] + gh[:, D:2 * D])
    n = jnp.tanh(gi[:, 2 * D:] + r * gh[:, 2 * D:])
    return (1.0 - z) * n + z * h


def _gru_body(parts_ref, h_ref, wiht_ref, whht_ref, bih_ref, bhh_ref,
              wgn_ref, hn_ref, mn_ref):
    hn = _gru_math(parts_ref, h_ref, wiht_ref, whht_ref, bih_ref, bhh_ref)
    hn_ref[...] = hn
    mn_ref[...] = jnp.dot(hn, wgn_ref[...], preferred_element_type=jnp.float32)


def _gru_out_body(parts_ref, h_ref, wiht_ref, whht_ref, bih_ref, bhh_ref,
                  w2t_ref, b2_ref, out_ref):
    hn = _gru_math(parts_ref, h_ref, wiht_ref, whht_ref, bih_ref, bhh_ref)
    hn = jnp.maximum(hn, 0.0)
    o = jnp.dot(hn, w2t_ref[...], preferred_element_type=jnp.float32)
    out_ref[...] = o + b2_ref[...]


def _row_spec(shape):
    nd = len(shape)
    if nd == 2:
        return pl.BlockSpec((BLK, shape[1]), lambda i: (i, 0))
    return pl.BlockSpec((shape[0], BLK, shape[2]), lambda i: (0, i, 0))


def _full_spec(shape):
    return pl.BlockSpec(shape, lambda i: tuple(0 for _ in shape))


def _tc_call(body, in_shapes, row_in, out_shapes):
    grid = (N // BLK,)
    in_specs = [_row_spec(s) if r else _full_spec(s)
                for s, r in zip(in_shapes, row_in)]
    out_specs = [_row_spec(s) for s in out_shapes]
    out_shape = [jax.ShapeDtypeStruct(s, jnp.float32) for s in out_shapes]
    if len(out_shapes) == 1:
        out_specs, out_shape = out_specs[0], out_shape[0]
    return pl.pallas_call(body, grid=grid, in_specs=in_specs,
                          out_specs=out_specs, out_shape=out_shape)


def kernel(x, edge_index, W1, b1, Wg, Wih, Whh, bih, bhh, W2, b2):
    w1t = W1.T
    wiht = Wih.T
    whht = Whh.T
    w2t = W2.T
    b1r = b1.reshape(1, D)
    bihr = bih.reshape(1, 3 * D)
    bhhr = bhh.reshape(1, 3 * D)
    b2r = b2.reshape(1, NT)

    src = edge_index[0]
    dst = edge_index[1]
    ecap0 = NS * CPT_A * K
    cap1 = NS * CPT_B * K
    pad1 = ecap0 + cap1 - E
    src0 = src[:ecap0].reshape(NS, CPT_A, K)
    dst0 = dst[:ecap0].reshape(NS, CPT_A, K)
    src1 = jnp.concatenate(
        [src[ecap0:], jnp.zeros((pad1,), jnp.int32)]).reshape(NS, CPT_B, K)
    dst1 = jnp.concatenate(
        [dst[ecap0:], jnp.full((pad1,), N, jnp.int32)]).reshape(NS, CPT_B, K)
    idx0 = jnp.stack([src0, dst0], axis=2)
    idx1 = jnp.stack([src1, dst1], axis=2)
    idx0 = jnp.pad(idx0, ((0, 0), (0, CPTMAX - CPT_A), (0, 0), (0, 0)))
    idx1 = jnp.pad(idx1, ((0, 0), (0, CPTMAX - CPT_B), (0, 0), (0, 0)))
    idx = jnp.stack([idx0, idx1], axis=0)
    zrow = jnp.zeros((K, D), jnp.float32)

    pre = _tc_call(_pre_body,
                   [(N, D), (D, D), (1, D), (D, D)],
                   [True, False, False, False],
                   [(N, D), (N, D)])
    h, m = pre(x, w1t, b1r, Wg[0])

    gru = _tc_call(_gru_body,
                   [(NC, NPAD, D), (N, D), (D, 3 * D), (D, 3 * D),
                    (1, 3 * D), (1, 3 * D), (D, D)],
                   [True, True, False, False, False, False, False],
                   [(N, D), (N, D)])
    gru_out = _tc_call(_gru_out_body,
                       [(NC, NPAD, D), (N, D), (D, 3 * D), (D, 3 * D),
                        (1, 3 * D), (1, 3 * D), (D, NT), (1, NT)],
                       [True, True, False, False, False, False, False, False],
                       [(N, NT)])

    for i in range(L):
        parts = _sc_scatter(idx, m, zrow)
        if i < L - 1:
            h, m = gru(parts, h, wiht, whht, bihr, bhhr, Wg[i + 1])
        else:
            out = gru_out(parts, h, wiht, whht, bihr, bhhr, w2t, b2r)
    return out.reshape(N, SURF, CEMB)
